# Initial kernel scaffold; baseline (speedup 1.0000x reference)
import jax
import jax.numpy as jnp
from jax.experimental import pallas as pl


def kernel(indices, task_ids, main_table, task_table):
    raise NotImplementedError("write your pallas kernel here")


# speedup vs baseline: 6.8282x; 6.8282x over previous
import jax
import jax.numpy as jnp
from jax import lax
from jax.experimental import pallas as pl
from jax.experimental.pallas import tpu as pltpu
from jax.experimental.pallas import tpu_sc as plsc

D = 16
F = 26
L = 20
B = 4096
TASK_DIM = 128
PER_ROW = F * L
OUT_D = F * D + TASK_DIM

NC, NS = 2, 16
NW = NC * NS
B_W = B // NW
C = 8
N_CHUNK = B_W // C
IDX_CHUNK = C * PER_ROW


def _sc_body(indices_hbm, task_ids_hbm, table_hbm, task_table_hbm, out_hbm,
             idx_v, rows_v, out_v, tids_v, task_rows_v, sem):
    wid = lax.axis_index("s") * NC + lax.axis_index("c")
    woff_rows = wid * B_W
    woff_idx = woff_rows * PER_ROW

    pltpu.sync_copy(task_ids_hbm.at[pl.ds(woff_rows, B_W)], tids_v)
    pltpu.async_copy(task_table_hbm.at[tids_v], task_rows_v, sem).wait()

    def chunk_body(g, carry):
        row_base = woff_rows + g * C
        pltpu.sync_copy(
            indices_hbm.at[pl.ds(woff_idx + g * IDX_CHUNK, IDX_CHUNK)], idx_v)
        pltpu.async_copy(table_hbm.at[idx_v], rows_v, sem).wait()
        for c in range(C):
            def field_body(f, carry2):
                base = c * PER_ROW + f * L
                acc = rows_v[base]
                for l in range(1, L):
                    acc = acc + rows_v[base + l]
                out_v[c, pl.ds(f * D, D)] = acc
                return carry2
            lax.fori_loop(0, F, field_body, 0)
            trow = g * C + c
            for r in range(TASK_DIM // 16):
                out_v[c, pl.ds(F * D + r * 16, 16)] = \
                    task_rows_v[trow, pl.ds(r * 16, 16)]
        pltpu.sync_copy(out_v, out_hbm.at[pl.ds(row_base, C), :])
        return carry

    lax.fori_loop(0, N_CHUNK, chunk_body, 0)


def kernel(indices, task_ids, main_table, task_table):
    idx_flat = indices.reshape(-1)
    mesh = plsc.VectorSubcoreMesh(core_axis_name="c", subcore_axis_name="s")
    run = pl.kernel(
        _sc_body,
        mesh=mesh,
        compiler_params=pltpu.CompilerParams(use_tc_tiling_on_sc=False),
        out_type=jax.ShapeDtypeStruct((B, OUT_D), jnp.float32),
        scratch_types=[
            pltpu.VMEM((IDX_CHUNK,), jnp.int32),
            pltpu.VMEM((IDX_CHUNK, D), jnp.float32),
            pltpu.VMEM((C, OUT_D), jnp.float32),
            pltpu.VMEM((B_W,), jnp.int32),
            pltpu.VMEM((B_W, TASK_DIM), jnp.float32),
            pltpu.SemaphoreType.DMA,
        ],
    )
    return run(idx_flat, task_ids, main_table, task_table)


# speedup vs baseline: 6.8282x; 1.0000x over previous
import jax
import jax.numpy as jnp
from jax import lax
from jax.experimental import pallas as pl
from jax.experimental.pallas import tpu as pltpu
from jax.experimental.pallas import tpu_sc as plsc

D = 16
F = 26
L = 20
B = 4096
TASK_DIM = 128
PER_ROW = F * L
OUT_D = F * D + TASK_DIM

NC, NS = 2, 16
NW = NC * NS
B_W = B // NW
C = 8
N_CHUNK = B_W // C
IDX_CHUNK = C * PER_ROW


def _sc_body(indices_hbm, task_ids_hbm, table_hbm, task_table_hbm, out_hbm,
             idx_v, rows_v, out_v, tids_v, task_rows_v, sem):
    wid = lax.axis_index("s") * NC + lax.axis_index("c")
    woff_rows = wid * B_W
    woff_idx = woff_rows * PER_ROW

    pltpu.sync_copy(task_ids_hbm.at[pl.ds(woff_rows, B_W)], tids_v)
    pltpu.async_copy(task_table_hbm.at[tids_v], task_rows_v, sem).wait()

    def chunk_body(g, carry):
        row_base = woff_rows + g * C
        pltpu.sync_copy(
            indices_hbm.at[pl.ds(woff_idx + g * IDX_CHUNK, IDX_CHUNK)], idx_v)
        pltpu.async_copy(table_hbm.at[idx_v], rows_v, sem).wait()
        for c in range(C):
            def field_body(f, carry2):
                base = c * PER_ROW + f * L
                acc = rows_v[base]
                for l in range(1, L):
                    acc = acc + rows_v[base + l]
                out_v[pl.ds(c * OUT_D + f * D, D)] = acc
                return carry2
            lax.fori_loop(0, F, field_body, 0)
            trow = g * C + c
            for r in range(TASK_DIM // 16):
                out_v[pl.ds(c * OUT_D + F * D + r * 16, 16)] = \
                    task_rows_v[trow, pl.ds(r * 16, 16)]
        pltpu.sync_copy(out_v, out_hbm.at[pl.ds(row_base * OUT_D, C * OUT_D)])
        return carry

    lax.fori_loop(0, N_CHUNK, chunk_body, 0)


def kernel(indices, task_ids, main_table, task_table):
    idx_flat = indices.reshape(-1)
    mesh = plsc.VectorSubcoreMesh(core_axis_name="c", subcore_axis_name="s")
    run = pl.kernel(
        _sc_body,
        mesh=mesh,
        compiler_params=pltpu.CompilerParams(use_tc_tiling_on_sc=False),
        out_type=jax.ShapeDtypeStruct((B * OUT_D,), jnp.float32),
        scratch_types=[
            pltpu.VMEM((IDX_CHUNK,), jnp.int32),
            pltpu.VMEM((IDX_CHUNK, D), jnp.float32),
            pltpu.VMEM((C * OUT_D,), jnp.float32),
            pltpu.VMEM((B_W,), jnp.int32),
            pltpu.VMEM((B_W, TASK_DIM), jnp.float32),
            pltpu.SemaphoreType.DMA,
        ],
    )
    return run(idx_flat, task_ids, main_table, task_table).reshape(B, OUT_D)
